# baseline (speedup 1.0000x reference)
import functools

import jax
import jax.numpy as jnp
from jax import lax
from jax.experimental import pallas as pl
from jax.experimental.pallas import tpu as pltpu
from jax.experimental.pallas import tpu_sc as plsc

_N, _E, _D = 10000, 320000, 128
_NC, _NS = 2, 16
_NW = _NC * _NS
_EPW = _E // _NW
_K = 80
_NB = _EPW // _K
_NBC = 25
_NCH = _NB // _NBC
_NR = 3
_ZR = 80
_NZC = _N // _ZR
_ZQ = -(-_NZC // _NS)


def _sc_gather_segsum(x, snd3, rcv3):
    mesh = plsc.VectorSubcoreMesh(core_axis_name="c", subcore_axis_name="s")

    @functools.partial(
        pl.kernel,
        out_type=jax.ShapeDtypeStruct((_NC, _N, _D), jnp.float32),
        mesh=mesh,
        scratch_types=[
            pltpu.VMEM((_NBC, _K), jnp.int32),
            pltpu.VMEM((2, _NBC, _K), jnp.int32),
            pltpu.VMEM((_NR, _K, _D), jnp.float32),
            pltpu.VMEM_SHARED((_N, _D), jnp.float32),
            pltpu.SemaphoreType.DMA((_NR, 2)),
            pltpu.SemaphoreType.DMA((_NR,)),
            pltpu.SemaphoreType.DMA,
        ],
    )
    def k(x_hbm, snd_hbm, rcv_hbm, out_hbm,
          snd_v, rcv_v, bufs, agg_sh, gsem, ssem, zsem):
        c = lax.axis_index("c")
        s = lax.axis_index("s")
        wid = s * _NC + c

        _KH = _K // 2

        def g_start(j, b):
            pltpu.async_copy(x_hbm.at[snd_v.at[j, pl.ds(0, _KH)]],
                             bufs.at[b, pl.ds(0, _KH)], gsem.at[b, 0])
            pltpu.async_copy(x_hbm.at[snd_v.at[j, pl.ds(_KH, _KH)]],
                             bufs.at[b, pl.ds(_KH, _KH)], gsem.at[b, 1])

        def g_wait(j, b):
            pltpu.make_async_copy(
                x_hbm.at[snd_v.at[j, pl.ds(0, _KH)]],
                bufs.at[b, pl.ds(0, _KH)], gsem.at[b, 0]).wait()
            pltpu.make_async_copy(
                x_hbm.at[snd_v.at[j, pl.ds(_KH, _KH)]],
                bufs.at[b, pl.ds(_KH, _KH)], gsem.at[b, 1]).wait()

        def s_start(j, b, e):
            pltpu.async_copy(bufs.at[b], agg_sh.at[rcv_v.at[e, j]],
                             ssem.at[b], add=True)

        def s_wait_global(g):
            b = lax.rem(g, _NR)
            jj = lax.rem(g, _NBC)
            ee = lax.rem(g // _NBC, 2)
            pltpu.make_async_copy(
                bufs.at[b], agg_sh.at[rcv_v.at[ee, jj]], ssem.at[b]).wait()

        @pl.loop(0, _ZR)
        def _(r):
            @pl.loop(0, _D, step=16)
            def _(cc):
                bufs[0, r, pl.ds(cc, 16)] = jnp.zeros((16,), jnp.float32)

        @pl.loop(0, _ZQ)
        def _(q):
            t = q * _NS + s

            @pl.when(t < _NZC)
            def _():
                pltpu.async_copy(bufs.at[0], agg_sh.at[pl.ds(t * _ZR, _ZR)],
                                 zsem)

        @pl.loop(0, _ZQ)
        def _(q):
            t = q * _NS + s

            @pl.when(t < _NZC)
            def _():
                pltpu.make_async_copy(
                    bufs.at[0], agg_sh.at[pl.ds(t * _ZR, _ZR)], zsem).wait()

        pltpu.sync_copy(snd_hbm.at[wid, 0], snd_v)
        pltpu.sync_copy(rcv_hbm.at[wid, 0], rcv_v.at[0])

        @pl.loop(0, _NR)
        def _(p):
            g_start(p, p)

        plsc.subcore_barrier()

        @pl.loop(0, _NCH)
        def _(h):
            e = lax.rem(h, 2)

            @pl.when(h > 0)
            def _():
                pltpu.sync_copy(snd_hbm.at[wid, h], snd_v)
                pltpu.sync_copy(rcv_hbm.at[wid, h], rcv_v.at[e])

            @pl.loop(0, _NBC + 2)
            def _(p):
                g = h * _NBC + p

                @pl.when(p < _NBC)
                def _():
                    @pl.when(g >= _NR)
                    def _():
                        s_wait_global(g - _NR)
                        g_start(p, lax.rem(g, _NR))

                @pl.when(p >= 2)
                def _():
                    q = p - 2
                    bq = lax.rem(g - 2, _NR)
                    g_wait(q, bq)
                    s_start(q, bq, e)

        @pl.loop(_NB - _NR, _NB)
        def _(g):
            s_wait_global(g)

        plsc.subcore_barrier()

        @pl.loop(0, _ZQ)
        def _(q):
            t = q * _NS + s

            @pl.when(t < _NZC)
            def _():
                pltpu.async_copy(agg_sh.at[pl.ds(t * _ZR, _ZR)],
                                 out_hbm.at[c, pl.ds(t * _ZR, _ZR)], zsem)

        @pl.loop(0, _ZQ)
        def _(q):
            t = q * _NS + s

            @pl.when(t < _NZC)
            def _():
                pltpu.make_async_copy(
                    agg_sh.at[pl.ds(t * _ZR, _ZR)],
                    out_hbm.at[c, pl.ds(t * _ZR, _ZR)], zsem).wait()

    return k(x, snd3, rcv3)


def _dense_proj_body(agg_ref, x_ref, w_ref, b_ref, r_ref, rb_ref, o_ref):
    a = agg_ref[0] + agg_ref[1]
    h = jnp.dot(a, w_ref[...], preferred_element_type=jnp.float32,
                precision=lax.Precision.HIGHEST)
    h = jnp.maximum(h + b_ref[...], 0.0)
    res = jnp.dot(x_ref[...], r_ref[...], preferred_element_type=jnp.float32,
                  precision=lax.Precision.HIGHEST)
    o_ref[...] = h + res + rb_ref[...]


def _dense_id_body(agg_ref, x_ref, w_ref, b_ref, o_ref):
    a = agg_ref[0] + agg_ref[1]
    h = jnp.dot(a, w_ref[...], preferred_element_type=jnp.float32,
                precision=lax.Precision.HIGHEST)
    o_ref[...] = jnp.maximum(h + b_ref[...], 0.0) + x_ref[...]


_GB = 2
_BM = _N // _GB


def _dense_proj(agg2, x, w, b, r, rb):
    return pl.pallas_call(
        _dense_proj_body,
        out_shape=jax.ShapeDtypeStruct((_N, _D), jnp.float32),
        grid=(_GB,),
        in_specs=[
            pl.BlockSpec((_NC, _BM, _D), lambda i: (0, i, 0)),
            pl.BlockSpec((_BM, _D), lambda i: (i, 0)),
            pl.BlockSpec((_D, _D), lambda i: (0, 0)),
            pl.BlockSpec((1, _D), lambda i: (0, 0)),
            pl.BlockSpec((_D, _D), lambda i: (0, 0)),
            pl.BlockSpec((1, _D), lambda i: (0, 0)),
        ],
        out_specs=pl.BlockSpec((_BM, _D), lambda i: (i, 0)),
    )(agg2, x, w, b, r, rb)


def _dense_id(agg2, x, w, b):
    return pl.pallas_call(
        _dense_id_body,
        out_shape=jax.ShapeDtypeStruct((_N, _D), jnp.float32),
        grid=(_GB,),
        in_specs=[
            pl.BlockSpec((_NC, _BM, _D), lambda i: (0, i, 0)),
            pl.BlockSpec((_BM, _D), lambda i: (i, 0)),
            pl.BlockSpec((_D, _D), lambda i: (0, 0)),
            pl.BlockSpec((1, _D), lambda i: (0, 0)),
        ],
        out_specs=pl.BlockSpec((_BM, _D), lambda i: (i, 0)),
    )(agg2, x, w, b)


def kernel(x, senders, receivers, W0, b0, W1, b1, W2, b2, R0, rb0, R2, rb2):
    snd3 = senders.reshape(_NW, _NCH, _NBC, _K)
    rcv3 = receivers.reshape(_NW, _NCH, _NBC, _K)
    b0r, b1r, b2r = b0.reshape(1, _D), b1.reshape(1, _D), b2.reshape(1, _D)
    rb0r, rb2r = rb0.reshape(1, _D), rb2.reshape(1, _D)

    agg = _sc_gather_segsum(x, snd3, rcv3)
    x1 = _dense_proj(agg, x, W0, b0r, R0, rb0r)
    agg = _sc_gather_segsum(x1, snd3, rcv3)
    x2 = _dense_id(agg, x1, W1, b1r)
    agg = _sc_gather_segsum(x2, snd3, rcv3)
    return _dense_proj(agg, x2, W2, b2r, R2, rb2r)


# speedup vs baseline: 1.0535x; 1.0535x over previous
import functools

import jax
import jax.numpy as jnp
from jax import lax
from jax.experimental import pallas as pl
from jax.experimental.pallas import tpu as pltpu
from jax.experimental.pallas import tpu_sc as plsc

_N, _E, _D = 10000, 320000, 128
_NC, _NS = 2, 16
_NW = _NC * _NS
_EPW = _E // _NW
_K = 80
_NB = _EPW // _K
_NBC = 25
_NCH = _NB // _NBC
_NR = 3
_ZR = 80
_NZC = _N // _ZR
_ZQ = -(-_NZC // _NS)


def _sc_gather_segsum(x, snd3, rcv3):
    mesh = plsc.VectorSubcoreMesh(core_axis_name="c", subcore_axis_name="s")

    @functools.partial(
        pl.kernel,
        out_type=jax.ShapeDtypeStruct((_NC, _N, _D), jnp.float32),
        mesh=mesh,
        scratch_types=[
            pltpu.VMEM((_NBC, _K), jnp.int32),
            pltpu.VMEM((2, _NBC, _K), jnp.int32),
            pltpu.VMEM((_NR, _K, _D), jnp.float32),
            pltpu.VMEM_SHARED((_N, _D), jnp.float32),
            pltpu.SemaphoreType.DMA((_NR, 2)),
            pltpu.SemaphoreType.DMA((_NR,)),
            pltpu.SemaphoreType.DMA,
        ],
    )
    def k(x_hbm, snd_hbm, rcv_hbm, out_hbm,
          snd_v, rcv_v, bufs, agg_sh, gsem, ssem, zsem):
        c = lax.axis_index("c")
        s = lax.axis_index("s")
        wid = s * _NC + c

        _KH = _K // 2

        def g_start(j, b):
            pltpu.async_copy(x_hbm.at[snd_v.at[j, pl.ds(0, _KH)]],
                             bufs.at[b, pl.ds(0, _KH)], gsem.at[b, 0])
            pltpu.async_copy(x_hbm.at[snd_v.at[j, pl.ds(_KH, _KH)]],
                             bufs.at[b, pl.ds(_KH, _KH)], gsem.at[b, 1])

        def g_wait(j, b):
            pltpu.make_async_copy(
                x_hbm.at[snd_v.at[j, pl.ds(0, _KH)]],
                bufs.at[b, pl.ds(0, _KH)], gsem.at[b, 0]).wait()
            pltpu.make_async_copy(
                x_hbm.at[snd_v.at[j, pl.ds(_KH, _KH)]],
                bufs.at[b, pl.ds(_KH, _KH)], gsem.at[b, 1]).wait()

        def s_start(j, b, e):
            pltpu.async_copy(bufs.at[b], agg_sh.at[rcv_v.at[e, j]],
                             ssem.at[b], add=True)

        def s_wait_global(g):
            b = lax.rem(g, _NR)
            jj = lax.rem(g, _NBC)
            ee = lax.rem(g // _NBC, 2)
            pltpu.make_async_copy(
                bufs.at[b], agg_sh.at[rcv_v.at[ee, jj]], ssem.at[b]).wait()

        @pl.loop(0, _ZR)
        def _(r):
            @pl.loop(0, _D, step=16)
            def _(cc):
                bufs[0, r, pl.ds(cc, 16)] = jnp.zeros((16,), jnp.float32)

        @pl.loop(0, _ZQ)
        def _(q):
            t = q * _NS + s

            @pl.when(t < _NZC)
            def _():
                pltpu.async_copy(bufs.at[0], agg_sh.at[pl.ds(t * _ZR, _ZR)],
                                 zsem)

        @pl.loop(0, _ZQ)
        def _(q):
            t = q * _NS + s

            @pl.when(t < _NZC)
            def _():
                pltpu.make_async_copy(
                    bufs.at[0], agg_sh.at[pl.ds(t * _ZR, _ZR)], zsem).wait()

        pltpu.sync_copy(snd_hbm.at[wid, 0], snd_v)
        pltpu.sync_copy(rcv_hbm.at[wid, 0], rcv_v.at[0])

        @pl.loop(0, _NR)
        def _(p):
            g_start(p, p)

        plsc.subcore_barrier()

        @pl.loop(0, _NCH)
        def _(h):
            e = lax.rem(h, 2)

            @pl.when(h > 0)
            def _():
                pltpu.sync_copy(snd_hbm.at[wid, h], snd_v)
                pltpu.sync_copy(rcv_hbm.at[wid, h], rcv_v.at[e])

            @pl.loop(0, _NBC + 2)
            def _(p):
                g = h * _NBC + p

                @pl.when(p < _NBC)
                def _():
                    @pl.when(g >= _NR)
                    def _():
                        s_wait_global(g - _NR)
                        g_start(p, lax.rem(g, _NR))

                @pl.when(p >= 2)
                def _():
                    q = p - 2
                    bq = lax.rem(g - 2, _NR)
                    g_wait(q, bq)
                    s_start(q, bq, e)

        @pl.loop(_NB - _NR, _NB)
        def _(g):
            s_wait_global(g)

        plsc.subcore_barrier()

        @pl.loop(0, _ZQ)
        def _(q):
            t = q * _NS + s

            @pl.when(t < _NZC)
            def _():
                pltpu.async_copy(agg_sh.at[pl.ds(t * _ZR, _ZR)],
                                 out_hbm.at[c, pl.ds(t * _ZR, _ZR)], zsem)

        @pl.loop(0, _ZQ)
        def _(q):
            t = q * _NS + s

            @pl.when(t < _NZC)
            def _():
                pltpu.make_async_copy(
                    agg_sh.at[pl.ds(t * _ZR, _ZR)],
                    out_hbm.at[c, pl.ds(t * _ZR, _ZR)], zsem).wait()

    return k(x, snd3, rcv3)


def _resid_body(x_ref, r_ref, rb_ref, o_ref):
    o_ref[...] = jnp.dot(x_ref[...], r_ref[...],
                         preferred_element_type=jnp.float32,
                         precision=lax.Precision.HIGHEST) + rb_ref[...]


def _dense_proj_body(agg_ref, res_ref, w_ref, b_ref, o_ref):
    a = agg_ref[0] + agg_ref[1]
    h = jnp.dot(a, w_ref[...], preferred_element_type=jnp.float32,
                precision=lax.Precision.HIGHEST)
    o_ref[...] = jnp.maximum(h + b_ref[...], 0.0) + res_ref[...]


def _dense_id_body(agg_ref, x_ref, w_ref, b_ref, o_ref):
    a = agg_ref[0] + agg_ref[1]
    h = jnp.dot(a, w_ref[...], preferred_element_type=jnp.float32,
                precision=lax.Precision.HIGHEST)
    o_ref[...] = jnp.maximum(h + b_ref[...], 0.0) + x_ref[...]


_GB = 5
_BM = _N // _GB


def _resid(x, r, rb):
    return pl.pallas_call(
        _resid_body,
        out_shape=jax.ShapeDtypeStruct((_N, _D), jnp.float32),
        grid=(_GB,),
        in_specs=[
            pl.BlockSpec((_BM, _D), lambda i: (i, 0)),
            pl.BlockSpec((_D, _D), lambda i: (0, 0)),
            pl.BlockSpec((1, _D), lambda i: (0, 0)),
        ],
        out_specs=pl.BlockSpec((_BM, _D), lambda i: (i, 0)),
    )(x, r, rb)


def _dense_proj(agg2, res, w, b):
    return pl.pallas_call(
        _dense_proj_body,
        out_shape=jax.ShapeDtypeStruct((_N, _D), jnp.float32),
        grid=(_GB,),
        in_specs=[
            pl.BlockSpec((_NC, _BM, _D), lambda i: (0, i, 0)),
            pl.BlockSpec((_BM, _D), lambda i: (i, 0)),
            pl.BlockSpec((_D, _D), lambda i: (0, 0)),
            pl.BlockSpec((1, _D), lambda i: (0, 0)),
        ],
        out_specs=pl.BlockSpec((_BM, _D), lambda i: (i, 0)),
    )(agg2, res, w, b)


def _dense_id(agg2, x, w, b):
    return pl.pallas_call(
        _dense_id_body,
        out_shape=jax.ShapeDtypeStruct((_N, _D), jnp.float32),
        grid=(_GB,),
        in_specs=[
            pl.BlockSpec((_NC, _BM, _D), lambda i: (0, i, 0)),
            pl.BlockSpec((_BM, _D), lambda i: (i, 0)),
            pl.BlockSpec((_D, _D), lambda i: (0, 0)),
            pl.BlockSpec((1, _D), lambda i: (0, 0)),
        ],
        out_specs=pl.BlockSpec((_BM, _D), lambda i: (i, 0)),
    )(agg2, x, w, b)


def kernel(x, senders, receivers, W0, b0, W1, b1, W2, b2, R0, rb0, R2, rb2):
    snd3 = senders.reshape(_NW, _NCH, _NBC, _K)
    rcv3 = receivers.reshape(_NW, _NCH, _NBC, _K)
    b0r, b1r, b2r = b0.reshape(1, _D), b1.reshape(1, _D), b2.reshape(1, _D)
    rb0r, rb2r = rb0.reshape(1, _D), rb2.reshape(1, _D)

    agg = _sc_gather_segsum(x, snd3, rcv3)
    res0 = _resid(x, R0, rb0r)
    x1 = _dense_proj(agg, res0, W0, b0r)
    agg = _sc_gather_segsum(x1, snd3, rcv3)
    x2 = _dense_id(agg, x1, W1, b1r)
    agg = _sc_gather_segsum(x2, snd3, rcv3)
    res2 = _resid(x2, R2, rb2r)
    return _dense_proj(agg, res2, W2, b2r)


# speedup vs baseline: 1.0717x; 1.0173x over previous
import functools

import jax
import jax.numpy as jnp
from jax import lax
from jax.experimental import pallas as pl
from jax.experimental.pallas import tpu as pltpu
from jax.experimental.pallas import tpu_sc as plsc

_N, _E, _D = 10000, 320000, 128
_NC, _NS = 2, 16
_NW = _NC * _NS
_EPW = _E // _NW
_K = 80
_NB = _EPW // _K
_NBC = 25
_NCH = _NB // _NBC
_NR = 3
_ZR = 80
_NZC = _N // _ZR
_ZQ = -(-_NZC // _NS)


def _sc_gather_segsum(x, snd3, rcv3):
    mesh = plsc.VectorSubcoreMesh(core_axis_name="c", subcore_axis_name="s")

    @functools.partial(
        pl.kernel,
        out_type=jax.ShapeDtypeStruct((_NC, _N, _D), jnp.float32),
        mesh=mesh,
        scratch_types=[
            pltpu.VMEM((_NBC, _K), jnp.int32),
            pltpu.VMEM((2, _NBC, _K), jnp.int32),
            pltpu.VMEM((_NR, _K, _D), jnp.float32),
            pltpu.VMEM_SHARED((_N, _D), jnp.float32),
            pltpu.SemaphoreType.DMA((_NR, 2)),
            pltpu.SemaphoreType.DMA((_NR,)),
            pltpu.SemaphoreType.DMA,
        ],
    )
    def k(x_hbm, snd_hbm, rcv_hbm, out_hbm,
          snd_v, rcv_v, bufs, agg_sh, gsem, ssem, zsem):
        c = lax.axis_index("c")
        s = lax.axis_index("s")
        wid = s * _NC + c

        _KH = _K // 2

        def g_start(j, b):
            pltpu.async_copy(x_hbm.at[snd_v.at[j, pl.ds(0, _KH)]],
                             bufs.at[b, pl.ds(0, _KH)], gsem.at[b, 0])
            pltpu.async_copy(x_hbm.at[snd_v.at[j, pl.ds(_KH, _KH)]],
                             bufs.at[b, pl.ds(_KH, _KH)], gsem.at[b, 1])

        def g_wait(j, b):
            pltpu.make_async_copy(
                x_hbm.at[snd_v.at[j, pl.ds(0, _KH)]],
                bufs.at[b, pl.ds(0, _KH)], gsem.at[b, 0]).wait()
            pltpu.make_async_copy(
                x_hbm.at[snd_v.at[j, pl.ds(_KH, _KH)]],
                bufs.at[b, pl.ds(_KH, _KH)], gsem.at[b, 1]).wait()

        def s_start(j, b, e):
            pltpu.async_copy(bufs.at[b], agg_sh.at[rcv_v.at[e, j]],
                             ssem.at[b], add=True)

        def s_wait_global(g):
            b = lax.rem(g, _NR)
            jj = lax.rem(g, _NBC)
            ee = lax.rem(g // _NBC, 2)
            pltpu.make_async_copy(
                bufs.at[b], agg_sh.at[rcv_v.at[ee, jj]], ssem.at[b]).wait()

        @pl.loop(0, _ZR)
        def _(r):
            @pl.loop(0, _D, step=16)
            def _(cc):
                bufs[0, r, pl.ds(cc, 16)] = jnp.zeros((16,), jnp.float32)

        @pl.loop(0, _ZQ)
        def _(q):
            t = q * _NS + s

            @pl.when(t < _NZC)
            def _():
                pltpu.async_copy(bufs.at[0], agg_sh.at[pl.ds(t * _ZR, _ZR)],
                                 zsem)

        pltpu.sync_copy(snd_hbm.at[wid, 0], snd_v)
        pltpu.sync_copy(rcv_hbm.at[wid, 0], rcv_v.at[0])

        @pl.loop(1, _NR)
        def _(p):
            g_start(p, p)

        @pl.loop(0, _ZQ)
        def _(q):
            t = q * _NS + s

            @pl.when(t < _NZC)
            def _():
                pltpu.make_async_copy(
                    bufs.at[0], agg_sh.at[pl.ds(t * _ZR, _ZR)], zsem).wait()

        g_start(0, 0)

        plsc.subcore_barrier()

        @pl.loop(0, _NCH)
        def _(h):
            e = lax.rem(h, 2)

            @pl.when(h > 0)
            def _():
                pltpu.sync_copy(snd_hbm.at[wid, h], snd_v)
                pltpu.sync_copy(rcv_hbm.at[wid, h], rcv_v.at[e])

            @pl.loop(0, _NBC + 2)
            def _(p):
                g = h * _NBC + p

                @pl.when(p < _NBC)
                def _():
                    @pl.when(g >= _NR)
                    def _():
                        s_wait_global(g - _NR)
                        g_start(p, lax.rem(g, _NR))

                @pl.when(p >= 2)
                def _():
                    q = p - 2
                    bq = lax.rem(g - 2, _NR)
                    g_wait(q, bq)
                    s_start(q, bq, e)

        @pl.loop(_NB - _NR, _NB)
        def _(g):
            s_wait_global(g)

        plsc.subcore_barrier()

        @pl.loop(0, _ZQ)
        def _(q):
            t = q * _NS + s

            @pl.when(t < _NZC)
            def _():
                pltpu.async_copy(agg_sh.at[pl.ds(t * _ZR, _ZR)],
                                 out_hbm.at[c, pl.ds(t * _ZR, _ZR)], zsem)

        @pl.loop(0, _ZQ)
        def _(q):
            t = q * _NS + s

            @pl.when(t < _NZC)
            def _():
                pltpu.make_async_copy(
                    agg_sh.at[pl.ds(t * _ZR, _ZR)],
                    out_hbm.at[c, pl.ds(t * _ZR, _ZR)], zsem).wait()

    return k(x, snd3, rcv3)


def _resid_body(x_ref, r_ref, rb_ref, o_ref):
    o_ref[...] = jnp.dot(x_ref[...], r_ref[...],
                         preferred_element_type=jnp.float32,
                         precision=lax.Precision.HIGHEST) + rb_ref[...]


def _dense_proj_body(agg_ref, res_ref, w_ref, b_ref, o_ref):
    a = agg_ref[0] + agg_ref[1]
    h = jnp.dot(a, w_ref[...], preferred_element_type=jnp.float32,
                precision=lax.Precision.HIGHEST)
    o_ref[...] = jnp.maximum(h + b_ref[...], 0.0) + res_ref[...]


def _dense_id_body(agg_ref, x_ref, w_ref, b_ref, o_ref):
    a = agg_ref[0] + agg_ref[1]
    h = jnp.dot(a, w_ref[...], preferred_element_type=jnp.float32,
                precision=lax.Precision.HIGHEST)
    o_ref[...] = jnp.maximum(h + b_ref[...], 0.0) + x_ref[...]


_GB = 5
_BM = _N // _GB


def _resid(x, r, rb):
    return pl.pallas_call(
        _resid_body,
        out_shape=jax.ShapeDtypeStruct((_N, _D), jnp.float32),
        grid=(_GB,),
        in_specs=[
            pl.BlockSpec((_BM, _D), lambda i: (i, 0)),
            pl.BlockSpec((_D, _D), lambda i: (0, 0)),
            pl.BlockSpec((1, _D), lambda i: (0, 0)),
        ],
        out_specs=pl.BlockSpec((_BM, _D), lambda i: (i, 0)),
    )(x, r, rb)


def _dense_proj(agg2, res, w, b):
    return pl.pallas_call(
        _dense_proj_body,
        out_shape=jax.ShapeDtypeStruct((_N, _D), jnp.float32),
        grid=(_GB,),
        in_specs=[
            pl.BlockSpec((_NC, _BM, _D), lambda i: (0, i, 0)),
            pl.BlockSpec((_BM, _D), lambda i: (i, 0)),
            pl.BlockSpec((_D, _D), lambda i: (0, 0)),
            pl.BlockSpec((1, _D), lambda i: (0, 0)),
        ],
        out_specs=pl.BlockSpec((_BM, _D), lambda i: (i, 0)),
    )(agg2, res, w, b)


def _dense_id(agg2, x, w, b):
    return pl.pallas_call(
        _dense_id_body,
        out_shape=jax.ShapeDtypeStruct((_N, _D), jnp.float32),
        grid=(_GB,),
        in_specs=[
            pl.BlockSpec((_NC, _BM, _D), lambda i: (0, i, 0)),
            pl.BlockSpec((_BM, _D), lambda i: (i, 0)),
            pl.BlockSpec((_D, _D), lambda i: (0, 0)),
            pl.BlockSpec((1, _D), lambda i: (0, 0)),
        ],
        out_specs=pl.BlockSpec((_BM, _D), lambda i: (i, 0)),
    )(agg2, x, w, b)


def kernel(x, senders, receivers, W0, b0, W1, b1, W2, b2, R0, rb0, R2, rb2):
    snd3 = senders.reshape(_NW, _NCH, _NBC, _K)
    rcv3 = receivers.reshape(_NW, _NCH, _NBC, _K)
    b0r, b1r, b2r = b0.reshape(1, _D), b1.reshape(1, _D), b2.reshape(1, _D)
    rb0r, rb2r = rb0.reshape(1, _D), rb2.reshape(1, _D)

    agg = _sc_gather_segsum(x, snd3, rcv3)
    res0 = _resid(x, R0, rb0r)
    x1 = _dense_proj(agg, res0, W0, b0r)
    agg = _sc_gather_segsum(x1, snd3, rcv3)
    x2 = _dense_id(agg, x1, W1, b1r)
    agg = _sc_gather_segsum(x2, snd3, rcv3)
    res2 = _resid(x2, R2, rb2r)
    return _dense_proj(agg, res2, W2, b2r)
